# Initial kernel scaffold; baseline (speedup 1.0000x reference)
import jax
import jax.numpy as jnp
from jax.experimental import pallas as pl


def kernel(embs, fw_adj_0, fw_adj_1, bw_adj_0, bw_adj_1, W_fw, b_fw, W_bw, b_bw, W_lin, b_lin):
    raise NotImplementedError("write your pallas kernel here")


# speedup vs baseline: 1.0877x; 1.0877x over previous
import jax
import jax.numpy as jnp
from jax.experimental import pallas as pl
from jax.experimental.pallas import tpu as pltpu

N = 4096
D = 512
H = 256
L = 2

BI = 1024
BK = 512
NI = N // BI
NK = N // BK


def _proj_kernel(h_ref, w_ref, out_ref):
    out_ref[...] = jnp.dot(h_ref[...], w_ref[...],
                           preferred_element_type=jnp.float32)


def _agg_kernel(bw0_ref, bw1_ref, fw0_ref, fw1_ref, xw_ref, wl_ref,
                bpre_ref, blin_ref, h_ref, out_ref, acc_ref):
    k = pl.program_id(1)

    @pl.when(k == 0)
    def _init():
        acc_ref[...] = jnp.zeros_like(acc_ref)

    xw = xw_ref[...]
    acc_ref[:, :H] += (
        jnp.dot(bw0_ref[...], xw[:, 0:H], preferred_element_type=jnp.float32)
        + jnp.dot(bw1_ref[...], xw[:, H:2 * H], preferred_element_type=jnp.float32))
    acc_ref[:, H:] += (
        jnp.dot(fw0_ref[...], xw[:, 2 * H:3 * H], preferred_element_type=jnp.float32)
        + jnp.dot(fw1_ref[...], xw[:, 3 * H:4 * H], preferred_element_type=jnp.float32))

    @pl.when(k == NK - 1)
    def _finalize():
        s = jnp.maximum(acc_ref[...] + bpre_ref[...], 0.0)
        out_ref[...] = (jnp.dot(s, wl_ref[...], preferred_element_type=jnp.float32)
                        + blin_ref[...] + h_ref[...])


def _bigcn_layer(h, bw0, bw1, fw0, fw1, Wcat, wl, bpre, blin):
    xw = pl.pallas_call(
        _proj_kernel,
        grid=(NI,),
        in_specs=[pl.BlockSpec((BI, D), lambda i: (i, 0)),
                  pl.BlockSpec((D, 4 * H), lambda i: (0, 0))],
        out_specs=pl.BlockSpec((BI, 4 * H), lambda i: (i, 0)),
        out_shape=jax.ShapeDtypeStruct((N, 4 * H), jnp.float32),
        compiler_params=pltpu.CompilerParams(
            dimension_semantics=("arbitrary",)),
    )(h, Wcat)

    adj_spec = pl.BlockSpec((BI, BK), lambda i, k: (i, k))
    out = pl.pallas_call(
        _agg_kernel,
        grid=(NI, NK),
        in_specs=[adj_spec, adj_spec, adj_spec, adj_spec,
                  pl.BlockSpec((BK, 4 * H), lambda i, k: (k, 0)),
                  pl.BlockSpec((D, D), lambda i, k: (0, 0)),
                  pl.BlockSpec((1, D), lambda i, k: (0, 0)),
                  pl.BlockSpec((1, D), lambda i, k: (0, 0)),
                  pl.BlockSpec((BI, D), lambda i, k: (i, 0))],
        out_specs=pl.BlockSpec((BI, D), lambda i, k: (i, 0)),
        out_shape=jax.ShapeDtypeStruct((N, D), jnp.float32),
        scratch_shapes=[pltpu.VMEM((BI, D), jnp.float32)],
        compiler_params=pltpu.CompilerParams(
            dimension_semantics=("parallel", "arbitrary")),
    )(bw0, bw1, fw0, fw1, xw, wl, bpre, blin, h)
    return out


def kernel(embs, fw_adj_0, fw_adj_1, bw_adj_0, bw_adj_1,
           W_fw, b_fw, W_bw, b_bw, W_lin, b_lin):
    h = embs
    for l in range(L):
        Wcat = jnp.concatenate(
            [W_bw[l, 0], W_bw[l, 1], W_fw[l, 0], W_fw[l, 1]], axis=1)
        bpre = jnp.concatenate(
            [b_bw[l, 0] + b_bw[l, 1], b_fw[l, 0] + b_fw[l, 1]])[None, :]
        blin = b_lin[l][None, :]
        h = _bigcn_layer(h, bw_adj_0, bw_adj_1, fw_adj_0, fw_adj_1,
                         Wcat, W_lin[l], bpre, blin)
    return h


# speedup vs baseline: 1.1936x; 1.0974x over previous
import jax
import jax.numpy as jnp
from jax.experimental import pallas as pl
from jax.experimental.pallas import tpu as pltpu

N = 4096
D = 512
H = 256
L = 2

BI = 1024
BK = 512
NI = N // BI
NK = N // BK


def _proj_kernel(h_ref, w_ref, out_ref):
    out_ref[...] = jnp.dot(h_ref[...], w_ref[...],
                           preferred_element_type=jnp.float32
                           ).astype(jnp.bfloat16)


def _agg_kernel(bw0_ref, bw1_ref, fw0_ref, fw1_ref, xw_ref, wl_ref,
                bpre_ref, blin_ref, h_ref, out_ref, acc_ref):
    k = pl.program_id(1)

    @pl.when(k == 0)
    def _init():
        acc_ref[...] = jnp.zeros_like(acc_ref)

    xw = xw_ref[...]
    bw0 = bw0_ref[...].astype(jnp.bfloat16)
    bw1 = bw1_ref[...].astype(jnp.bfloat16)
    fw0 = fw0_ref[...].astype(jnp.bfloat16)
    fw1 = fw1_ref[...].astype(jnp.bfloat16)
    acc_ref[:, :H] += (
        jnp.dot(bw0, xw[:, 0:H], preferred_element_type=jnp.float32)
        + jnp.dot(bw1, xw[:, H:2 * H], preferred_element_type=jnp.float32))
    acc_ref[:, H:] += (
        jnp.dot(fw0, xw[:, 2 * H:3 * H], preferred_element_type=jnp.float32)
        + jnp.dot(fw1, xw[:, 3 * H:4 * H], preferred_element_type=jnp.float32))

    @pl.when(k == NK - 1)
    def _finalize():
        s = jnp.maximum(acc_ref[...] + bpre_ref[...], 0.0)
        out_ref[...] = (jnp.dot(s, wl_ref[...], preferred_element_type=jnp.float32)
                        + blin_ref[...] + h_ref[...])


def _bigcn_layer(h, bw0, bw1, fw0, fw1, Wcat, wl, bpre, blin):
    xw = pl.pallas_call(
        _proj_kernel,
        grid=(NI,),
        in_specs=[pl.BlockSpec((BI, D), lambda i: (i, 0)),
                  pl.BlockSpec((D, 4 * H), lambda i: (0, 0))],
        out_specs=pl.BlockSpec((BI, 4 * H), lambda i: (i, 0)),
        out_shape=jax.ShapeDtypeStruct((N, 4 * H), jnp.bfloat16),
        compiler_params=pltpu.CompilerParams(
            dimension_semantics=("arbitrary",)),
    )(h, Wcat)

    adj_spec = pl.BlockSpec((BI, BK), lambda i, k: (i, k))
    out = pl.pallas_call(
        _agg_kernel,
        grid=(NI, NK),
        in_specs=[adj_spec, adj_spec, adj_spec, adj_spec,
                  pl.BlockSpec((BK, 4 * H), lambda i, k: (k, 0)),
                  pl.BlockSpec((D, D), lambda i, k: (0, 0)),
                  pl.BlockSpec((1, D), lambda i, k: (0, 0)),
                  pl.BlockSpec((1, D), lambda i, k: (0, 0)),
                  pl.BlockSpec((BI, D), lambda i, k: (i, 0))],
        out_specs=pl.BlockSpec((BI, D), lambda i, k: (i, 0)),
        out_shape=jax.ShapeDtypeStruct((N, D), jnp.float32),
        scratch_shapes=[pltpu.VMEM((BI, D), jnp.float32)],
        compiler_params=pltpu.CompilerParams(
            dimension_semantics=("parallel", "arbitrary")),
    )(bw0, bw1, fw0, fw1, xw, wl, bpre, blin, h)
    return out


def kernel(embs, fw_adj_0, fw_adj_1, bw_adj_0, bw_adj_1,
           W_fw, b_fw, W_bw, b_bw, W_lin, b_lin):
    h = embs
    for l in range(L):
        Wcat = jnp.concatenate(
            [W_bw[l, 0], W_bw[l, 1], W_fw[l, 0], W_fw[l, 1]], axis=1)
        bpre = jnp.concatenate(
            [b_bw[l, 0] + b_bw[l, 1], b_fw[l, 0] + b_fw[l, 1]])[None, :]
        blin = b_lin[l][None, :]
        h = _bigcn_layer(h, bw_adj_0, bw_adj_1, fw_adj_0, fw_adj_1,
                         Wcat, W_lin[l], bpre, blin)
    return h


# speedup vs baseline: 1.2210x; 1.0230x over previous
import jax
import jax.numpy as jnp
from jax.experimental import pallas as pl
from jax.experimental.pallas import tpu as pltpu

N = 4096
D = 512
H = 256
L = 2

BI = 2048
BK = 256
NI = N // BI
NK = N // BK


def _proj_kernel(h_ref, w_ref, out_ref):
    out_ref[...] = jnp.dot(h_ref[...], w_ref[...],
                           preferred_element_type=jnp.float32
                           ).astype(jnp.bfloat16)


def _agg_kernel(bw0_ref, bw1_ref, fw0_ref, fw1_ref, xw_ref, wl_ref,
                bpre_ref, blin_ref, h_ref, out_ref, acc_ref):
    k = pl.program_id(1)

    @pl.when(k == 0)
    def _init():
        acc_ref[...] = jnp.zeros_like(acc_ref)

    xw = xw_ref[...]
    bw0 = bw0_ref[...].astype(jnp.bfloat16)
    bw1 = bw1_ref[...].astype(jnp.bfloat16)
    fw0 = fw0_ref[...].astype(jnp.bfloat16)
    fw1 = fw1_ref[...].astype(jnp.bfloat16)
    acc_ref[:, :H] += (
        jnp.dot(bw0, xw[:, 0:H], preferred_element_type=jnp.float32)
        + jnp.dot(bw1, xw[:, H:2 * H], preferred_element_type=jnp.float32))
    acc_ref[:, H:] += (
        jnp.dot(fw0, xw[:, 2 * H:3 * H], preferred_element_type=jnp.float32)
        + jnp.dot(fw1, xw[:, 3 * H:4 * H], preferred_element_type=jnp.float32))

    @pl.when(k == NK - 1)
    def _finalize():
        s = jnp.maximum(acc_ref[...] + bpre_ref[...], 0.0)
        out_ref[...] = (jnp.dot(s, wl_ref[...], preferred_element_type=jnp.float32)
                        + blin_ref[...] + h_ref[...])


def _bigcn_layer(h, bw0, bw1, fw0, fw1, Wcat, wl, bpre, blin):
    xw = pl.pallas_call(
        _proj_kernel,
        grid=(NI,),
        in_specs=[pl.BlockSpec((BI, D), lambda i: (i, 0)),
                  pl.BlockSpec((D, 4 * H), lambda i: (0, 0))],
        out_specs=pl.BlockSpec((BI, 4 * H), lambda i: (i, 0)),
        out_shape=jax.ShapeDtypeStruct((N, 4 * H), jnp.bfloat16),
        compiler_params=pltpu.CompilerParams(
            dimension_semantics=("arbitrary",)),
    )(h, Wcat)

    adj_spec = pl.BlockSpec((BI, BK), lambda i, k: (i, k))
    out = pl.pallas_call(
        _agg_kernel,
        grid=(NI, NK),
        in_specs=[adj_spec, adj_spec, adj_spec, adj_spec,
                  pl.BlockSpec((BK, 4 * H), lambda i, k: (k, 0)),
                  pl.BlockSpec((D, D), lambda i, k: (0, 0)),
                  pl.BlockSpec((1, D), lambda i, k: (0, 0)),
                  pl.BlockSpec((1, D), lambda i, k: (0, 0)),
                  pl.BlockSpec((BI, D), lambda i, k: (i, 0))],
        out_specs=pl.BlockSpec((BI, D), lambda i, k: (i, 0)),
        out_shape=jax.ShapeDtypeStruct((N, D), jnp.float32),
        scratch_shapes=[pltpu.VMEM((BI, D), jnp.float32)],
        compiler_params=pltpu.CompilerParams(
            dimension_semantics=("parallel", "arbitrary")),
    )(bw0, bw1, fw0, fw1, xw, wl, bpre, blin, h)
    return out


def kernel(embs, fw_adj_0, fw_adj_1, bw_adj_0, bw_adj_1,
           W_fw, b_fw, W_bw, b_bw, W_lin, b_lin):
    h = embs
    for l in range(L):
        Wcat = jnp.concatenate(
            [W_bw[l, 0], W_bw[l, 1], W_fw[l, 0], W_fw[l, 1]], axis=1)
        bpre = jnp.concatenate(
            [b_bw[l, 0] + b_bw[l, 1], b_fw[l, 0] + b_fw[l, 1]])[None, :]
        blin = b_lin[l][None, :]
        h = _bigcn_layer(h, bw_adj_0, bw_adj_1, fw_adj_0, fw_adj_1,
                         Wcat, W_lin[l], bpre, blin)
    return h
